# baseline (speedup 1.0000x reference)
import functools

import jax
import jax.numpy as jnp
from jax import lax
from jax.experimental import pallas as pl
from jax.experimental.pallas import tpu as pltpu
from jax.experimental.pallas import tpu_sc as plsc

Q_TOTAL = 1024
N_KEYS = 100000
DIM = 64
K_KEEP = 33
K_PAD = 48

QB = 64
CPAD = 100352
CW = 1024
NCH = CPAD // CW
NCHPAD = 128
MW = 2048
NMM = CPAD // MW
FOLD = 8
SW = CPAD // FOLD
S3R = QB * FOLD
PREP_B = 2048
NEG = float("-inf")
BIGCOL = 2**30

NW = 32
RPT = Q_TOTAL // NW
PERT = RPT * K_PAD


def _prep_body(xn_ref, out_ref):
    v = xn_ref[...]
    nrm = jnp.sqrt(jnp.sum(v * v, axis=1, keepdims=True))
    out_ref[...] = (v / jnp.maximum(nrm, 1e-12)).T


def _topk_body(q_ref, xnn_ref, vals_ref, cols_ref, s_ref, cm_ref, amc_ref,
               nm_ref, cselv_ref, csels_ref, sems):
    q = q_ref[...]
    qn = q / jnp.maximum(jnp.sqrt(jnp.sum(q * q, axis=1, keepdims=True)), 1e-12)

    cm_ref[...] = jnp.full((QB, NCHPAD), NEG, jnp.float32)
    vals_ref[...] = jnp.full((QB, K_PAD), NEG, jnp.float32)
    cols_ref[...] = jnp.zeros((QB, K_PAD), jnp.int32)

    mcol = jax.lax.broadcasted_iota(jnp.int32, (QB, MW), 1)
    li = jax.lax.broadcasted_iota(jnp.int32, (QB, NCHPAD), 1)
    kiota = jax.lax.broadcasted_iota(jnp.int32, (QB, K_PAD), 1)
    iota8 = (jax.lax.broadcasted_iota(jnp.int32, (FOLD, 128), 0) * 128
             + jax.lax.broadcasted_iota(jnp.int32, (FOLD, 128), 1))
    FCH = MW // (FOLD * 128)

    def mm(c, carry):
        off = pl.multiple_of(c * MW, MW)
        blk = xnn_ref[:, pl.ds(off, MW)]
        sim = jax.lax.dot_general(
            qn, blk, (((1,), (0,)), ((), ())),
            preferred_element_type=jnp.float32)
        sim = jnp.where(c * MW + mcol < N_KEYS, sim, NEG)
        folded = sim.reshape(QB, FCH, FOLD, 128).transpose(0, 2, 1, 3)
        s_ref[:, pl.ds(pl.multiple_of(c * (MW // FOLD), MW // FOLD),
                       MW // FOLD)] = folded.reshape(S3R, MW // FOLD)
        cmu = cm_ref[...]
        for sub in range(MW // CW):
            mx = jnp.max(sim[:, sub * CW:(sub + 1) * CW], axis=1,
                         keepdims=True)
            cmu = jnp.where(li == c * (MW // CW) + sub, mx, cmu)
        cm_ref[...] = cmu
        return carry

    lax.fori_loop(0, NMM, mm, 0, unroll=False)

    NSTR = 4 if QB >= 64 else 2
    SB = QB // NSTR
    liS = jax.lax.broadcasted_iota(jnp.int32, (SB, NCHPAD), 1)

    def stream_prefix(h):
        lo = h * SB
        cmv = cm_ref[lo:lo + SB, :]
        m = jnp.max(cmv, axis=1, keepdims=True)
        csel = jnp.min(jnp.where(cmv == m, liS, BIGCOL), axis=1,
                       keepdims=True)
        cselv_ref[lo:lo + SB, :] = csel
        pltpu.make_async_copy(cselv_ref.at[pl.ds(lo, SB)],
                              csels_ref.at[pl.ds(lo, SB)], sems.at[h]).start()
        return m, csel

    def stream_rows(h, m):
        lo = h * SB
        offs, chunks = [], []
        for k in range(SB):
            r = lo + k
            c_r = csels_ref[r, 0]
            off = pl.multiple_of(c_r * 128, 128)
            offs.append(off)
            chunks.append(s_ref[FOLD * r:FOLD * (r + 1), pl.ds(off, 128)])
        news, nms, amcs = [], [], []
        for k in range(SB):
            blkv = chunks[k]
            eq = blkv == m[k:k + 1, :]
            am = jnp.min(jnp.min(jnp.where(eq, iota8, BIGCOL), axis=1,
                                 keepdims=True), axis=0, keepdims=True)
            newblk = jnp.where(iota8 == am, NEG, blkv)
            news.append(newblk)
            nms.append(jnp.max(jnp.max(newblk, axis=1, keepdims=True),
                               axis=0, keepdims=True))
            amcs.append(am)
        for k in range(SB):
            r = lo + k
            s_ref[FOLD * r:FOLD * (r + 1), pl.ds(offs[k], 128)] = news[k]
        nm_ref[lo:lo + SB, :] = jnp.concatenate(nms, axis=0)
        amc_ref[lo:lo + SB, :] = jnp.concatenate(
            [offs[k] * FOLD + amcs[k] for k in range(SB)], axis=0)

    def stream_wait(h):
        lo = h * SB
        pltpu.make_async_copy(cselv_ref.at[pl.ds(lo, SB)],
                              csels_ref.at[pl.ds(lo, SB)], sems.at[h]).wait()

    carry0 = []
    for h in range(NSTR):
        carry0 += list(stream_prefix(h))

    def extract(j, carry):
        ms = list(carry[0::2])
        csels = list(carry[1::2])
        nxt = []
        for h in range(NSTR):
            lo = h * SB
            stream_wait(h)
            stream_rows(h, ms[h])
            cm_ref[lo:lo + SB, :] = jnp.where(
                liS == csels[h], nm_ref[lo:lo + SB, :], cm_ref[lo:lo + SB, :])
            nxt += list(stream_prefix(h))
        m = jnp.concatenate(ms, axis=0)
        vals_ref[...] = jnp.where(kiota == j, m, vals_ref[...])
        cols_ref[...] = jnp.where(kiota == j, amc_ref[...], cols_ref[...])
        return tuple(nxt)

    lax.fori_loop(0, K_KEEP, extract, tuple(carry0), unroll=False)
    for h in range(NSTR):
        stream_wait(h)

    vals_ref[...] = jnp.where(kiota >= K_KEEP, vals_ref[:, 0:1], vals_ref[...])
    cols_ref[...] = jnp.where(kiota >= K_KEEP, cols_ref[:, 0:1], cols_ref[...])


def _topk_call(x, xnn, interpret=False):
    return pl.pallas_call(
        _topk_body,
        grid=(Q_TOTAL // QB,),
        in_specs=[
            pl.BlockSpec((QB, DIM), lambda i: (i, 0)),
            pl.BlockSpec((DIM, CPAD), lambda i: (0, 0)),
        ],
        out_specs=[
            pl.BlockSpec((QB, K_PAD), lambda i: (i, 0)),
            pl.BlockSpec((QB, K_PAD), lambda i: (i, 0)),
        ],
        out_shape=[
            jax.ShapeDtypeStruct((Q_TOTAL, K_PAD), jnp.float32),
            jax.ShapeDtypeStruct((Q_TOTAL, K_PAD), jnp.int32),
        ],
        scratch_shapes=[
            pltpu.VMEM((S3R, SW), jnp.float32),
            pltpu.VMEM((QB, NCHPAD), jnp.float32),
            pltpu.VMEM((QB, 1), jnp.int32),
            pltpu.VMEM((QB, 1), jnp.float32),
            pltpu.VMEM((QB, 1), jnp.int32),
            pltpu.SMEM((QB, 1), jnp.int32),
            pltpu.SemaphoreType.DMA((4,)),
        ],
        interpret=interpret,
    )(x, xnn)


def _prep_call(x_n, interpret=False):
    xp = jnp.pad(x_n, ((0, CPAD - N_KEYS), (0, 0)))
    return pl.pallas_call(
        _prep_body,
        grid=(CPAD // PREP_B,),
        in_specs=[pl.BlockSpec((PREP_B, DIM), lambda i: (i, 0))],
        out_specs=pl.BlockSpec((DIM, PREP_B), lambda i: (0, i)),
        out_shape=jax.ShapeDtypeStruct((DIM, CPAD), jnp.float32),
        interpret=interpret,
    )(xp)


def _make_scatter_kernel():
    mesh = plsc.VectorSubcoreMesh(core_axis_name="c", subcore_axis_name="s")

    @functools.partial(
        pl.kernel,
        out_type=jax.ShapeDtypeStruct((Q_TOTAL, N_KEYS), jnp.float32),
        mesh=mesh,
        compiler_params=pltpu.CompilerParams(needs_layout_passes=False),
        scratch_types=[
            pltpu.VMEM((N_KEYS,), jnp.float32),
            pltpu.VMEM((PERT,), jnp.int32),
            pltpu.VMEM((PERT,), jnp.float32),
        ],
    )
    def scatter_kernel(vals_hbm, idx_hbm, out_hbm, row_v, idx_v, val_v):
        wid = lax.axis_index("s") * 2 + lax.axis_index("c")

        def fill_neg(i, carry):
            row_v[pl.ds(i * 16, 16)] = jnp.full((16,), NEG, jnp.float32)
            return carry

        lax.fori_loop(0, N_KEYS // 16, fill_neg, 0, unroll=False)

        pltpu.sync_copy(idx_hbm.at[wid], idx_v)
        pltpu.sync_copy(vals_hbm.at[wid], val_v)
        negv = jnp.full((16,), NEG, jnp.float32)
        for r in range(RPT):
            for k in range(K_PAD // 16):
                o = r * K_PAD + k * 16
                plsc.store_scatter(row_v, [idx_v[pl.ds(o, 16)]],
                                   val_v[pl.ds(o, 16)])
            pltpu.sync_copy(row_v, out_hbm.at[wid * RPT + r])
            for k in range(K_PAD // 16):
                o = r * K_PAD + k * 16
                plsc.store_scatter(row_v, [idx_v[pl.ds(o, 16)]], negv)

    return scatter_kernel


def kernel(x, x_n):
    xnn = _prep_call(x_n)
    vals, cols = _topk_call(x, xnn)
    vals2 = vals.reshape(NW, PERT)
    idx2 = cols.reshape(NW, PERT)
    out = _make_scatter_kernel()(vals2, idx2)
    return out


# speedup vs baseline: 1.2780x; 1.2780x over previous
import functools

import jax
import jax.numpy as jnp
from jax import lax
from jax.experimental import pallas as pl
from jax.experimental.pallas import tpu as pltpu
from jax.experimental.pallas import tpu_sc as plsc

Q_TOTAL = 1024
N_KEYS = 100000
DIM = 64
K_KEEP = 33
K_PAD = 48

QB = 64
CPAD = 100352
CW = 1024
NCH = CPAD // CW
NCHPAD = 128
MW = 2048
NMM = CPAD // MW
FOLD = 8
SW = CPAD // FOLD
S3R = QB * FOLD
PREP_B = 2048
NEG = float("-inf")
BIGCOL = 2**30

NW = 32
RPT = Q_TOTAL // NW
PERT = RPT * K_PAD


def _prep_body(xn_ref, out_ref):
    v = xn_ref[...]
    nrm = jnp.sqrt(jnp.sum(v * v, axis=1, keepdims=True))
    out_ref[...] = (v / jnp.maximum(nrm, 1e-12)).T


def _topk_body(q_ref, xnn_ref, vals_ref, cols_ref, s_ref, cm_ref, amc_ref,
               nm_ref, cselv_ref, csels_ref, sems):
    q = q_ref[...]
    qn = q / jnp.maximum(jnp.sqrt(jnp.sum(q * q, axis=1, keepdims=True)), 1e-12)

    cm_ref[...] = jnp.full((QB, NCHPAD), NEG, jnp.float32)
    vals_ref[...] = jnp.full((QB, K_PAD), NEG, jnp.float32)
    cols_ref[...] = jnp.zeros((QB, K_PAD), jnp.int32)

    mcol = jax.lax.broadcasted_iota(jnp.int32, (QB, MW), 1)
    li = jax.lax.broadcasted_iota(jnp.int32, (QB, NCHPAD), 1)
    kiota = jax.lax.broadcasted_iota(jnp.int32, (QB, K_PAD), 1)
    iota8 = (jax.lax.broadcasted_iota(jnp.int32, (FOLD, 128), 0) * 128
             + jax.lax.broadcasted_iota(jnp.int32, (FOLD, 128), 1))
    FCH = MW // (FOLD * 128)

    def mm(c, carry):
        off = pl.multiple_of(c * MW, MW)
        blk = xnn_ref[:, pl.ds(off, MW)]
        sim = jax.lax.dot_general(
            qn, blk, (((1,), (0,)), ((), ())),
            preferred_element_type=jnp.float32)
        sim = jnp.where(c * MW + mcol < N_KEYS, sim, NEG)
        folded = sim.reshape(QB, FCH, FOLD, 128).transpose(0, 2, 1, 3)
        s_ref[:, pl.ds(pl.multiple_of(c * (MW // FOLD), MW // FOLD),
                       MW // FOLD)] = folded.reshape(S3R, MW // FOLD)
        cmu = cm_ref[...]
        for sub in range(MW // CW):
            mx = jnp.max(sim[:, sub * CW:(sub + 1) * CW], axis=1,
                         keepdims=True)
            cmu = jnp.where(li == c * (MW // CW) + sub, mx, cmu)
        cm_ref[...] = cmu
        return carry

    lax.fori_loop(0, NMM, mm, 0, unroll=False)

    NSTR = 2
    SB = QB // NSTR
    liS = jax.lax.broadcasted_iota(jnp.int32, (SB, NCHPAD), 1)

    def stream_prefix(h):
        lo = h * SB
        cmv = cm_ref[lo:lo + SB, :]
        m = jnp.max(cmv, axis=1, keepdims=True)
        csel = jnp.min(jnp.where(cmv == m, liS, BIGCOL), axis=1,
                       keepdims=True)
        cselv_ref[lo:lo + SB, :] = csel
        pltpu.make_async_copy(cselv_ref.at[pl.ds(lo, SB)],
                              csels_ref.at[pl.ds(lo, SB)], sems.at[h]).start()
        return m, csel

    def stream_rows(h, m):
        lo = h * SB
        offs, chunks = [], []
        for k in range(SB):
            r = lo + k
            c_r = csels_ref[r, 0]
            off = pl.multiple_of(c_r * 128, 128)
            offs.append(off)
            chunks.append(s_ref[FOLD * r:FOLD * (r + 1), pl.ds(off, 128)])
        news, nms, amcs = [], [], []
        for k in range(SB):
            blkv = chunks[k]
            eq = blkv == m[k:k + 1, :]
            am = jnp.min(jnp.min(jnp.where(eq, iota8, BIGCOL), axis=1,
                                 keepdims=True), axis=0, keepdims=True)
            newblk = jnp.where(iota8 == am, NEG, blkv)
            news.append(newblk)
            nms.append(jnp.max(jnp.max(newblk, axis=1, keepdims=True),
                               axis=0, keepdims=True))
            amcs.append(am)
        for k in range(SB):
            r = lo + k
            s_ref[FOLD * r:FOLD * (r + 1), pl.ds(offs[k], 128)] = news[k]
        nm_ref[lo:lo + SB, :] = jnp.concatenate(nms, axis=0)
        amc_ref[lo:lo + SB, :] = jnp.concatenate(
            [offs[k] * FOLD + amcs[k] for k in range(SB)], axis=0)

    def stream_wait(h):
        lo = h * SB
        pltpu.make_async_copy(cselv_ref.at[pl.ds(lo, SB)],
                              csels_ref.at[pl.ds(lo, SB)], sems.at[h]).wait()

    carry0 = []
    for h in range(NSTR):
        carry0 += list(stream_prefix(h))

    def extract(j, carry):
        ms = list(carry[0::2])
        csels = list(carry[1::2])
        nxt = []
        for h in range(NSTR):
            lo = h * SB
            stream_wait(h)
            stream_rows(h, ms[h])
            cm_ref[lo:lo + SB, :] = jnp.where(
                liS == csels[h], nm_ref[lo:lo + SB, :], cm_ref[lo:lo + SB, :])
            nxt += list(stream_prefix(h))
        m = jnp.concatenate(ms, axis=0)
        vals_ref[...] = jnp.where(kiota == j, m, vals_ref[...])
        cols_ref[...] = jnp.where(kiota == j, amc_ref[...], cols_ref[...])
        return tuple(nxt)

    lax.fori_loop(0, K_KEEP, extract, tuple(carry0), unroll=False)
    for h in range(NSTR):
        stream_wait(h)

    vals_ref[...] = jnp.where(kiota >= K_KEEP, vals_ref[:, 0:1], vals_ref[...])
    cols_ref[...] = jnp.where(kiota >= K_KEEP, cols_ref[:, 0:1], cols_ref[...])


def _topk_call(x, xnn, interpret=False):
    return pl.pallas_call(
        _topk_body,
        grid=(Q_TOTAL // QB,),
        in_specs=[
            pl.BlockSpec((QB, DIM), lambda i: (i, 0)),
            pl.BlockSpec((DIM, CPAD), lambda i: (0, 0)),
        ],
        out_specs=[
            pl.BlockSpec((QB, K_PAD), lambda i: (i, 0)),
            pl.BlockSpec((QB, K_PAD), lambda i: (i, 0)),
        ],
        out_shape=[
            jax.ShapeDtypeStruct((Q_TOTAL, K_PAD), jnp.float32),
            jax.ShapeDtypeStruct((Q_TOTAL, K_PAD), jnp.int32),
        ],
        scratch_shapes=[
            pltpu.VMEM((S3R, SW), jnp.float32),
            pltpu.VMEM((QB, NCHPAD), jnp.float32),
            pltpu.VMEM((QB, 1), jnp.int32),
            pltpu.VMEM((QB, 1), jnp.float32),
            pltpu.VMEM((QB, 1), jnp.int32),
            pltpu.SMEM((QB, 1), jnp.int32),
            pltpu.SemaphoreType.DMA((4,)),
        ],
        interpret=interpret,
    )(x, xnn)


def _prep_call(x_n, interpret=False):
    xp = jnp.pad(x_n, ((0, CPAD - N_KEYS), (0, 0)))
    return pl.pallas_call(
        _prep_body,
        grid=(CPAD // PREP_B,),
        in_specs=[pl.BlockSpec((PREP_B, DIM), lambda i: (i, 0))],
        out_specs=pl.BlockSpec((DIM, PREP_B), lambda i: (0, i)),
        out_shape=jax.ShapeDtypeStruct((DIM, CPAD), jnp.float32),
        interpret=interpret,
    )(xp)


def _make_scatter_kernel():
    mesh = plsc.VectorSubcoreMesh(core_axis_name="c", subcore_axis_name="s")

    @functools.partial(
        pl.kernel,
        out_type=jax.ShapeDtypeStruct((Q_TOTAL, N_KEYS), jnp.float32),
        mesh=mesh,
        compiler_params=pltpu.CompilerParams(needs_layout_passes=False),
        scratch_types=[
            pltpu.VMEM((N_KEYS,), jnp.float32),
            pltpu.VMEM((PERT,), jnp.int32),
            pltpu.VMEM((PERT,), jnp.float32),
        ],
    )
    def scatter_kernel(vals_hbm, idx_hbm, out_hbm, row_v, idx_v, val_v):
        wid = lax.axis_index("s") * 2 + lax.axis_index("c")

        def fill_neg(i, carry):
            row_v[pl.ds(i * 16, 16)] = jnp.full((16,), NEG, jnp.float32)
            return carry

        lax.fori_loop(0, N_KEYS // 16, fill_neg, 0, unroll=False)

        pltpu.sync_copy(idx_hbm.at[wid], idx_v)
        pltpu.sync_copy(vals_hbm.at[wid], val_v)
        negv = jnp.full((16,), NEG, jnp.float32)
        for r in range(RPT):
            for k in range(K_PAD // 16):
                o = r * K_PAD + k * 16
                plsc.store_scatter(row_v, [idx_v[pl.ds(o, 16)]],
                                   val_v[pl.ds(o, 16)])
            pltpu.sync_copy(row_v, out_hbm.at[wid * RPT + r])
            for k in range(K_PAD // 16):
                o = r * K_PAD + k * 16
                plsc.store_scatter(row_v, [idx_v[pl.ds(o, 16)]], negv)

    return scatter_kernel


def kernel(x, x_n):
    xnn = _prep_call(x_n)
    vals, cols = _topk_call(x, xnn)
    vals2 = vals.reshape(NW, PERT)
    idx2 = cols.reshape(NW, PERT)
    out = _make_scatter_kernel()(vals2, idx2)
    return out


# speedup vs baseline: 1.4223x; 1.1129x over previous
import functools

import jax
import jax.numpy as jnp
from jax import lax
from jax.experimental import pallas as pl
from jax.experimental.pallas import tpu as pltpu
from jax.experimental.pallas import tpu_sc as plsc

Q_TOTAL = 1024
N_KEYS = 100000
DIM = 64
K_KEEP = 33
K_PAD = 48

QB = 64
CPAD = 100352
CW = 1024
NCH = CPAD // CW
NCHPAD = 128
MW = 2048
NMM = CPAD // MW
FOLD = 8
SW = CPAD // FOLD
S3R = QB * FOLD
PREP_B = 2048
NEG = float("-inf")
BIGCOL = 2**30

NW = 32
RPT = Q_TOTAL // NW
PERT = RPT * K_PAD


def _prep_body(xn_ref, out_ref):
    v = xn_ref[...]
    nrm = jnp.sqrt(jnp.sum(v * v, axis=1, keepdims=True))
    out_ref[...] = (v / jnp.maximum(nrm, 1e-12)).T


def _topk_body(q_ref, xnn_ref, vals_ref, cols_ref, s_ref, cm_ref, amc_ref,
               nm_ref, cselv_ref, csels_ref, sems):
    q = q_ref[...]
    qn = q / jnp.maximum(jnp.sqrt(jnp.sum(q * q, axis=1, keepdims=True)), 1e-12)

    cm_ref[...] = jnp.full((QB, NCHPAD), NEG, jnp.float32)
    vals_ref[...] = jnp.full((QB, K_PAD), NEG, jnp.float32)
    cols_ref[...] = jnp.zeros((QB, K_PAD), jnp.int32)

    mcol = jax.lax.broadcasted_iota(jnp.int32, (QB, MW), 1)
    li = jax.lax.broadcasted_iota(jnp.int32, (QB, NCHPAD), 1)
    kiota = jax.lax.broadcasted_iota(jnp.int32, (QB, K_PAD), 1)
    iota8 = (jax.lax.broadcasted_iota(jnp.int32, (FOLD, 128), 0) * 128
             + jax.lax.broadcasted_iota(jnp.int32, (FOLD, 128), 1))
    FCH = MW // (FOLD * 128)

    def mm(c, carry):
        off = pl.multiple_of(c * MW, MW)
        blk = xnn_ref[:, pl.ds(off, MW)]
        sim = jax.lax.dot_general(
            qn, blk, (((1,), (0,)), ((), ())),
            preferred_element_type=jnp.float32)
        sim = jnp.where(c * MW + mcol < N_KEYS, sim, NEG)
        for f in range(FCH):
            s_ref[:, pl.ds(pl.multiple_of(c * (MW // FOLD) + f * 128, 128),
                           128)] = (
                sim[:, f * FOLD * 128:(f + 1) * FOLD * 128].reshape(S3R, 128))
        cmu = cm_ref[...]
        for sub in range(MW // CW):
            mx = jnp.max(sim[:, sub * CW:(sub + 1) * CW], axis=1,
                         keepdims=True)
            cmu = jnp.where(li == c * (MW // CW) + sub, mx, cmu)
        cm_ref[...] = cmu
        return carry

    lax.fori_loop(0, NMM, mm, 0, unroll=False)

    NSTR = 2
    SB = QB // NSTR
    liS = jax.lax.broadcasted_iota(jnp.int32, (SB, NCHPAD), 1)

    def stream_prefix(h):
        lo = h * SB
        cmv = cm_ref[lo:lo + SB, :]
        m = jnp.max(cmv, axis=1, keepdims=True)
        csel = jnp.min(jnp.where(cmv == m, liS, BIGCOL), axis=1,
                       keepdims=True)
        cselv_ref[lo:lo + SB, :] = csel
        pltpu.make_async_copy(cselv_ref.at[pl.ds(lo, SB)],
                              csels_ref.at[pl.ds(lo, SB)], sems.at[h]).start()
        return m, csel

    def stream_rows(h, m):
        lo = h * SB
        offs, chunks = [], []
        for k in range(SB):
            r = lo + k
            c_r = csels_ref[r, 0]
            off = pl.multiple_of(c_r * 128, 128)
            offs.append(off)
            chunks.append(s_ref[FOLD * r:FOLD * (r + 1), pl.ds(off, 128)])
        news, nms, amcs = [], [], []
        for k in range(SB):
            blkv = chunks[k]
            eq = blkv == m[k:k + 1, :]
            am = jnp.min(jnp.min(jnp.where(eq, iota8, BIGCOL), axis=1,
                                 keepdims=True), axis=0, keepdims=True)
            newblk = jnp.where(iota8 == am, NEG, blkv)
            news.append(newblk)
            nms.append(jnp.max(jnp.max(newblk, axis=1, keepdims=True),
                               axis=0, keepdims=True))
            amcs.append(am)
        for k in range(SB):
            r = lo + k
            s_ref[FOLD * r:FOLD * (r + 1), pl.ds(offs[k], 128)] = news[k]
        nm_ref[lo:lo + SB, :] = jnp.concatenate(nms, axis=0)
        amc_ref[lo:lo + SB, :] = jnp.concatenate(
            [offs[k] * FOLD + amcs[k] for k in range(SB)], axis=0)

    def stream_wait(h):
        lo = h * SB
        pltpu.make_async_copy(cselv_ref.at[pl.ds(lo, SB)],
                              csels_ref.at[pl.ds(lo, SB)], sems.at[h]).wait()

    carry0 = []
    for h in range(NSTR):
        carry0 += list(stream_prefix(h))

    def extract(j, carry):
        ms = list(carry[0::2])
        csels = list(carry[1::2])
        nxt = []
        for h in range(NSTR):
            lo = h * SB
            stream_wait(h)
            stream_rows(h, ms[h])
            cm_ref[lo:lo + SB, :] = jnp.where(
                liS == csels[h], nm_ref[lo:lo + SB, :], cm_ref[lo:lo + SB, :])
            nxt += list(stream_prefix(h))
        m = jnp.concatenate(ms, axis=0)
        vals_ref[...] = jnp.where(kiota == j, m, vals_ref[...])
        cols_ref[...] = jnp.where(kiota == j, amc_ref[...], cols_ref[...])
        return tuple(nxt)

    lax.fori_loop(0, K_KEEP, extract, tuple(carry0), unroll=False)
    for h in range(NSTR):
        stream_wait(h)

    vals_ref[...] = jnp.where(kiota >= K_KEEP, vals_ref[:, 0:1], vals_ref[...])
    cols_ref[...] = jnp.where(kiota >= K_KEEP, cols_ref[:, 0:1], cols_ref[...])


def _topk_call(x, xnn, interpret=False):
    return pl.pallas_call(
        _topk_body,
        grid=(Q_TOTAL // QB,),
        in_specs=[
            pl.BlockSpec((QB, DIM), lambda i: (i, 0)),
            pl.BlockSpec((DIM, CPAD), lambda i: (0, 0)),
        ],
        out_specs=[
            pl.BlockSpec((QB, K_PAD), lambda i: (i, 0)),
            pl.BlockSpec((QB, K_PAD), lambda i: (i, 0)),
        ],
        out_shape=[
            jax.ShapeDtypeStruct((Q_TOTAL, K_PAD), jnp.float32),
            jax.ShapeDtypeStruct((Q_TOTAL, K_PAD), jnp.int32),
        ],
        scratch_shapes=[
            pltpu.VMEM((S3R, SW), jnp.float32),
            pltpu.VMEM((QB, NCHPAD), jnp.float32),
            pltpu.VMEM((QB, 1), jnp.int32),
            pltpu.VMEM((QB, 1), jnp.float32),
            pltpu.VMEM((QB, 1), jnp.int32),
            pltpu.SMEM((QB, 1), jnp.int32),
            pltpu.SemaphoreType.DMA((4,)),
        ],
        interpret=interpret,
    )(x, xnn)


def _prep_call(x_n, interpret=False):
    xp = jnp.pad(x_n, ((0, CPAD - N_KEYS), (0, 0)))
    return pl.pallas_call(
        _prep_body,
        grid=(CPAD // PREP_B,),
        in_specs=[pl.BlockSpec((PREP_B, DIM), lambda i: (i, 0))],
        out_specs=pl.BlockSpec((DIM, PREP_B), lambda i: (0, i)),
        out_shape=jax.ShapeDtypeStruct((DIM, CPAD), jnp.float32),
        interpret=interpret,
    )(xp)


def _make_scatter_kernel():
    mesh = plsc.VectorSubcoreMesh(core_axis_name="c", subcore_axis_name="s")

    @functools.partial(
        pl.kernel,
        out_type=jax.ShapeDtypeStruct((Q_TOTAL, N_KEYS), jnp.float32),
        mesh=mesh,
        compiler_params=pltpu.CompilerParams(needs_layout_passes=False),
        scratch_types=[
            pltpu.VMEM((N_KEYS,), jnp.float32),
            pltpu.VMEM((PERT,), jnp.int32),
            pltpu.VMEM((PERT,), jnp.float32),
        ],
    )
    def scatter_kernel(vals_hbm, idx_hbm, out_hbm, row_v, idx_v, val_v):
        wid = lax.axis_index("s") * 2 + lax.axis_index("c")

        def fill_neg(i, carry):
            row_v[pl.ds(i * 16, 16)] = jnp.full((16,), NEG, jnp.float32)
            return carry

        lax.fori_loop(0, N_KEYS // 16, fill_neg, 0, unroll=False)

        pltpu.sync_copy(idx_hbm.at[wid], idx_v)
        pltpu.sync_copy(vals_hbm.at[wid], val_v)
        negv = jnp.full((16,), NEG, jnp.float32)
        for r in range(RPT):
            for k in range(K_PAD // 16):
                o = r * K_PAD + k * 16
                plsc.store_scatter(row_v, [idx_v[pl.ds(o, 16)]],
                                   val_v[pl.ds(o, 16)])
            pltpu.sync_copy(row_v, out_hbm.at[wid * RPT + r])
            for k in range(K_PAD // 16):
                o = r * K_PAD + k * 16
                plsc.store_scatter(row_v, [idx_v[pl.ds(o, 16)]], negv)

    return scatter_kernel


def kernel(x, x_n):
    xnn = _prep_call(x_n)
    vals, cols = _topk_call(x, xnn)
    vals2 = vals.reshape(NW, PERT)
    idx2 = cols.reshape(NW, PERT)
    out = _make_scatter_kernel()(vals2, idx2)
    return out
